# baseline (speedup 1.0000x reference)
import functools

import jax
import jax.numpy as jnp
from jax import lax
from jax.experimental import pallas as pl
from jax.experimental.pallas import tpu as pltpu
from jax.experimental.pallas import tpu_sc as plsc

BATCH, SEQ, D = 4, 8192, 1024
NUM_CORES, NUM_SUBCORES = 2, 16
NUM_WORKERS = NUM_CORES * NUM_SUBCORES
SEQ_PER_W = SEQ // NUM_WORKERS
CHUNK = 32
N_CHUNKS = SEQ_PER_W // CHUNK
LANES = 16
CHUNK_ELEMS = CHUNK * D
N_VREGS = CHUNK_ELEMS // LANES


def _pe_add_body(x_hbm, pos_hbm, out_hbm, pos_v, x_v):
    wid = lax.axis_index("s") * NUM_CORES + lax.axis_index("c")
    seq0 = wid * SEQ_PER_W

    def chunk_loop(c, _):
        base = (seq0 + c * CHUNK) * D
        pltpu.sync_copy(pos_hbm.at[pl.ds(base, CHUNK_ELEMS)], pos_v)

        def batch_loop(b, _):
            xoff = b * (SEQ * D) + base
            pltpu.sync_copy(x_hbm.at[pl.ds(xoff, CHUNK_ELEMS)], x_v)

            def add_loop(i, _):
                pv = pos_v[pl.ds(i * LANES, LANES)]
                plsc.addupdate(x_v.at[pl.ds(i * LANES, LANES)], pv)
                return 0

            lax.fori_loop(0, N_VREGS, add_loop, 0)
            pltpu.sync_copy(x_v, out_hbm.at[pl.ds(xoff, CHUNK_ELEMS)])
            return 0

        lax.fori_loop(0, BATCH, batch_loop, 0)
        return 0

    lax.fori_loop(0, N_CHUNKS, chunk_loop, 0)


@functools.partial(
    pl.kernel,
    mesh=plsc.VectorSubcoreMesh(core_axis_name="c", subcore_axis_name="s"),
    out_type=jax.ShapeDtypeStruct((BATCH * SEQ * D,), jnp.float32),
    scratch_types=[
        pltpu.VMEM((CHUNK_ELEMS,), jnp.float32),
        pltpu.VMEM((CHUNK_ELEMS,), jnp.float32),
    ],
)
def _pe_add(x_hbm, pos_hbm, out_hbm, pos_v, x_v):
    _pe_add_body(x_hbm, pos_hbm, out_hbm, pos_v, x_v)


def kernel(x, pos_table):
    xf = x.reshape(-1)
    pf = pos_table.reshape(-1)
    out = _pe_add(xf, pf)
    return out.reshape(x.shape)


# speedup vs baseline: 1.7318x; 1.7318x over previous
import functools

import jax
import jax.numpy as jnp
from jax import lax
from jax.experimental import pallas as pl
from jax.experimental.pallas import tpu as pltpu
from jax.experimental.pallas import tpu_sc as plsc

BATCH, SEQ, D = 4, 8192, 1024
NUM_CORES, NUM_SUBCORES = 2, 16
NUM_WORKERS = NUM_CORES * NUM_SUBCORES
SEQ_PER_W = SEQ // NUM_WORKERS
CHUNK = 16
N_CHUNKS = SEQ_PER_W // CHUNK
LANES = 16
CHUNK_ELEMS = CHUNK * D
N_VREGS = CHUNK_ELEMS // LANES
UNROLL = 8


def _pe_add_body(x_hbm, pos_hbm, out_hbm,
                 pos0, pos1, xb0, xb1, xb2, xb3,
                 sp0, sp1, si0, si1, si2, si3, so0, so1, so2, so3):
    pos_bufs = (pos0, pos1)
    pos_sems = (sp0, sp1)
    x_bufs = (xb0, xb1, xb2, xb3)
    in_sems = (si0, si1, si2, si3)
    out_sems = (so0, so1, so2, so3)

    wid = lax.axis_index("s") * NUM_CORES + lax.axis_index("c")
    seq0 = wid * SEQ_PER_W

    def pos_cp(c, parity):
        base = (seq0 + c * CHUNK) * D
        return pltpu.make_async_copy(
            pos_hbm.at[pl.ds(base, CHUNK_ELEMS)], pos_bufs[parity],
            pos_sems[parity])

    def in_cp(c, b):
        off = (b * SEQ + seq0 + c * CHUNK) * D
        return pltpu.make_async_copy(
            x_hbm.at[pl.ds(off, CHUNK_ELEMS)], x_bufs[b], in_sems[b])

    def out_cp(c, b):
        off = (b * SEQ + seq0 + c * CHUNK) * D
        return pltpu.make_async_copy(
            x_bufs[b], out_hbm.at[pl.ds(off, CHUNK_ELEMS)], out_sems[b])

    def add_chunk(xb, posbuf):
        def body(i, _):
            base = i * (LANES * UNROLL)
            for u in range(UNROLL):
                off = base + u * LANES
                plsc.addupdate(xb.at[pl.ds(off, LANES)],
                               posbuf[pl.ds(off, LANES)])
            return 0
        lax.fori_loop(0, N_VREGS // UNROLL, body, 0)

    def chunk_step(c, parity):
        @pl.when(c + 1 < N_CHUNKS)
        def _():
            pos_cp(c + 1, parity ^ 1).start()

        pos_cp(c, parity).wait()
        for b in range(BATCH):
            in_cp(c, b).wait()
            add_chunk(x_bufs[b], pos_bufs[parity])
            out_cp(c, b).start()
            if b >= 1:
                out_cp(c, b - 1).wait()

                @pl.when(c + 1 < N_CHUNKS)
                def _():
                    in_cp(c + 1, b - 1).start()

        out_cp(c, BATCH - 1).wait()

        @pl.when(c + 1 < N_CHUNKS)
        def _():
            in_cp(c + 1, BATCH - 1).start()

    pos_cp(0, 0).start()
    for b in range(BATCH):
        in_cp(0, b).start()

    def loop_body(c2, _):
        chunk_step(c2 * 2, 0)
        chunk_step(c2 * 2 + 1, 1)
        return 0

    lax.fori_loop(0, N_CHUNKS // 2, loop_body, 0)


@functools.partial(
    pl.kernel,
    mesh=plsc.VectorSubcoreMesh(core_axis_name="c", subcore_axis_name="s"),
    out_type=jax.ShapeDtypeStruct((BATCH * SEQ * D,), jnp.float32),
    scratch_types=(
        [pltpu.VMEM((CHUNK_ELEMS,), jnp.float32)] * 2
        + [pltpu.VMEM((CHUNK_ELEMS,), jnp.float32)] * 4
        + [pltpu.SemaphoreType.DMA] * 10
    ),
)
def _pe_add(*refs):
    _pe_add_body(*refs)


def kernel(x, pos_table):
    xf = x.reshape(-1)
    pf = pos_table.reshape(-1)
    out = _pe_add(xf, pf)
    return out.reshape(x.shape)


# speedup vs baseline: 2.1580x; 1.2461x over previous
import functools

import jax
import jax.numpy as jnp
from jax import lax
from jax.experimental import pallas as pl
from jax.experimental.pallas import tpu as pltpu
from jax.experimental.pallas import tpu_sc as plsc

BATCH, SEQ, D = 4, 8192, 1024
NUM_CORES, NUM_SUBCORES = 2, 16
NUM_WORKERS = NUM_CORES * NUM_SUBCORES
SEQ_PER_W = SEQ // NUM_WORKERS
CHUNK = 16
N_CHUNKS = SEQ_PER_W // CHUNK
LANES = 16
CHUNK_ELEMS = CHUNK * D
N_VREGS = CHUNK_ELEMS // LANES
UNROLL = 8


def _pe_add_body(x_hbm, pos_hbm, out_hbm,
                 pos0, pos1, xb0, xb1, xb2, xb3,
                 sp0, sp1, si0, si1, si2, si3, so0, so1, so2, so3):
    pos_bufs = (pos0, pos1)
    pos_sems = (sp0, sp1)
    x_bufs = (xb0, xb1, xb2, xb3)
    in_sems = (si0, si1, si2, si3)
    out_sems = (so0, so1, so2, so3)

    wid = lax.axis_index("s") * NUM_CORES + lax.axis_index("c")
    seq0 = wid * SEQ_PER_W

    def pos_cp(c, parity):
        row0 = seq0 + c * CHUNK
        return pltpu.make_async_copy(
            pos_hbm.at[pl.ds(row0, CHUNK), :], pos_bufs[parity],
            pos_sems[parity])

    def in_cp(c, b):
        row0 = seq0 + c * CHUNK
        return pltpu.make_async_copy(
            x_hbm.at[b, pl.ds(row0, CHUNK), :], x_bufs[b], in_sems[b])

    def out_cp(c, b):
        row0 = seq0 + c * CHUNK
        return pltpu.make_async_copy(
            x_bufs[b], out_hbm.at[b, pl.ds(row0, CHUNK), :], out_sems[b])

    def add_chunk(xb, posbuf):
        def row_body(r, _):
            def col_body(j, _):
                base = j * (LANES * UNROLL)
                for u in range(UNROLL):
                    off = base + u * LANES
                    plsc.addupdate(xb.at[r, pl.ds(off, LANES)],
                                   posbuf[r, pl.ds(off, LANES)])
                return 0
            lax.fori_loop(0, D // (LANES * UNROLL), col_body, 0)
            return 0
        lax.fori_loop(0, CHUNK, row_body, 0)

    def chunk_step(c, parity):
        @pl.when(c + 1 < N_CHUNKS)
        def _():
            pos_cp(c + 1, parity ^ 1).start()

        pos_cp(c, parity).wait()
        for b in range(BATCH):
            in_cp(c, b).wait()
            add_chunk(x_bufs[b], pos_bufs[parity])
            out_cp(c, b).start()
            if b >= 1:
                out_cp(c, b - 1).wait()

                @pl.when(c + 1 < N_CHUNKS)
                def _():
                    in_cp(c + 1, b - 1).start()

        out_cp(c, BATCH - 1).wait()

        @pl.when(c + 1 < N_CHUNKS)
        def _():
            in_cp(c + 1, BATCH - 1).start()

    pos_cp(0, 0).start()
    for b in range(BATCH):
        in_cp(0, b).start()

    def loop_body(c2, _):
        chunk_step(c2 * 2, 0)
        chunk_step(c2 * 2 + 1, 1)
        return 0

    lax.fori_loop(0, N_CHUNKS // 2, loop_body, 0)


@functools.partial(
    pl.kernel,
    mesh=plsc.VectorSubcoreMesh(core_axis_name="c", subcore_axis_name="s"),
    out_type=jax.ShapeDtypeStruct((BATCH, SEQ, D), jnp.float32),
    scratch_types=(
        [pltpu.VMEM((CHUNK, D), jnp.float32)] * 2
        + [pltpu.VMEM((CHUNK, D), jnp.float32)] * 4
        + [pltpu.SemaphoreType.DMA] * 10
    ),
)
def _pe_add(*refs):
    _pe_add_body(*refs)


def kernel(x, pos_table):
    return _pe_add(x, pos_table)


# speedup vs baseline: 5.2182x; 2.4181x over previous
import functools

import jax
import jax.numpy as jnp
from jax import lax
from jax.experimental import pallas as pl
from jax.experimental.pallas import tpu as pltpu
from jax.experimental.pallas import tpu_sc as plsc

BATCH, SEQ, D = 4, 8192, 1024
NUM_CORES, NUM_SUBCORES = 2, 16
NUM_WORKERS = NUM_CORES * NUM_SUBCORES
SEQ_PER_W = SEQ // NUM_WORKERS
CHUNK = 16
N_CHUNKS = SEQ_PER_W // CHUNK
LANES = 16
CHUNK_ELEMS = CHUNK * D
N_VREGS = CHUNK_ELEMS // LANES
UNROLL = 8


def _pe_add_body(x_hbm, pos_hbm, out_hbm,
                 pos0, pos1, xb0, xb1, xb2, xb3,
                 sp0, sp1, si0, si1, si2, si3, so0, so1, so2, so3):
    pos_bufs = (pos0, pos1)
    pos_sems = (sp0, sp1)
    x_bufs = (xb0, xb1, xb2, xb3)
    in_sems = (si0, si1, si2, si3)
    out_sems = (so0, so1, so2, so3)

    wid = lax.axis_index("s") * NUM_CORES + lax.axis_index("c")
    seq0 = wid * SEQ_PER_W

    def start_pos(c, parity):
        row0 = seq0 + c * CHUNK

        def body(r, _):
            pltpu.async_copy(pos_hbm.at[row0 + r, :],
                             pos_bufs[parity].at[pl.ds(r * D, D)],
                             pos_sems[parity])
            return 0
        lax.fori_loop(0, CHUNK, body, 0)

    def wait_pos(parity):
        def body(r, _):
            pltpu.make_async_copy(pos_hbm.at[0, :],
                                  pos_bufs[parity].at[pl.ds(0, D)],
                                  pos_sems[parity]).wait()
            return 0
        lax.fori_loop(0, CHUNK, body, 0)

    def start_in(c, b):
        row0 = seq0 + c * CHUNK

        def body(r, _):
            pltpu.async_copy(x_hbm.at[b, row0 + r, :],
                             x_bufs[b].at[pl.ds(r * D, D)],
                             in_sems[b])
            return 0
        lax.fori_loop(0, CHUNK, body, 0)

    def wait_in(b):
        def body(r, _):
            pltpu.make_async_copy(x_hbm.at[b, 0, :],
                                  x_bufs[b].at[pl.ds(0, D)],
                                  in_sems[b]).wait()
            return 0
        lax.fori_loop(0, CHUNK, body, 0)

    def start_out(c, b):
        row0 = seq0 + c * CHUNK

        def body(r, _):
            pltpu.async_copy(x_bufs[b].at[pl.ds(r * D, D)],
                             out_hbm.at[b, row0 + r, :],
                             out_sems[b])
            return 0
        lax.fori_loop(0, CHUNK, body, 0)

    def wait_out_start_in(c_next, b):
        def body(r, _):
            pltpu.make_async_copy(x_bufs[b].at[pl.ds(0, D)],
                                  out_hbm.at[b, 0, :],
                                  out_sems[b]).wait()

            @pl.when(c_next < N_CHUNKS)
            def _():
                pltpu.async_copy(x_hbm.at[b, seq0 + c_next * CHUNK + r, :],
                                 x_bufs[b].at[pl.ds(r * D, D)],
                                 in_sems[b])
            return 0
        lax.fori_loop(0, CHUNK, body, 0)

    def add_chunk(xb, posbuf):
        def body(i, _):
            base = i * (LANES * UNROLL)
            for u in range(UNROLL):
                off = base + u * LANES
                plsc.addupdate(xb.at[pl.ds(off, LANES)],
                               posbuf[pl.ds(off, LANES)])
            return 0
        lax.fori_loop(0, N_VREGS // UNROLL, body, 0)

    def chunk_step(c, parity):
        @pl.when(c + 1 < N_CHUNKS)
        def _():
            start_pos(c + 1, parity ^ 1)

        wait_pos(parity)
        for b in range(BATCH):
            wait_in(b)
            add_chunk(x_bufs[b], pos_bufs[parity])
            start_out(c, b)
            if b >= 1:
                wait_out_start_in(c + 1, b - 1)
        wait_out_start_in(c + 1, BATCH - 1)

    start_pos(0, 0)
    for b in range(BATCH):
        start_in(0, b)

    def loop_body(c2, _):
        chunk_step(c2 * 2, 0)
        chunk_step(c2 * 2 + 1, 1)
        return 0

    lax.fori_loop(0, N_CHUNKS // 2, loop_body, 0)


@functools.partial(
    pl.kernel,
    mesh=plsc.VectorSubcoreMesh(core_axis_name="c", subcore_axis_name="s"),
    out_type=jax.ShapeDtypeStruct((BATCH, SEQ, D), jnp.float32),
    scratch_types=(
        [pltpu.VMEM((CHUNK_ELEMS,), jnp.float32)] * 2
        + [pltpu.VMEM((CHUNK_ELEMS,), jnp.float32)] * 4
        + [pltpu.SemaphoreType.DMA] * 10
    ),
)
def _pe_add(*refs):
    _pe_add_body(*refs)


def kernel(x, pos_table):
    return _pe_add(x, pos_table)


# speedup vs baseline: 5.2265x; 1.0016x over previous
import functools

import jax
import jax.numpy as jnp
from jax import lax
from jax.experimental import pallas as pl
from jax.experimental.pallas import tpu as pltpu
from jax.experimental.pallas import tpu_sc as plsc

BATCH, SEQ, D = 4, 8192, 1024
NUM_CORES, NUM_SUBCORES = 2, 16
NUM_WORKERS = NUM_CORES * NUM_SUBCORES
SEQ_PER_W = SEQ // NUM_WORKERS
CHUNK = 16
N_CHUNKS = SEQ_PER_W // CHUNK
LANES = 16
CHUNK_ELEMS = CHUNK * D
N_VREGS = CHUNK_ELEMS // LANES
UNROLL = 8


def _pe_add_body(x_hbm, pos_hbm, out_hbm,
                 pos0, pos1, xb0, xb1, xb2, xb3,
                 sp0, sp1, si0, si1, si2, si3, so0, so1, so2, so3):
    pos_bufs = (pos0, pos1)
    pos_sems = (sp0, sp1)
    x_bufs = (xb0, xb1, xb2, xb3)
    in_sems = (si0, si1, si2, si3)
    out_sems = (so0, so1, so2, so3)

    wid = lax.axis_index("s") * NUM_CORES + lax.axis_index("c")
    seq0 = wid * SEQ_PER_W

    def start_pos(c, parity):
        row0 = seq0 + c * CHUNK

        def body(r, _):
            pltpu.async_copy(pos_hbm.at[row0 + r, :],
                             pos_bufs[parity].at[pl.ds(r * D, D)],
                             pos_sems[parity])
            return 0
        lax.fori_loop(0, CHUNK, body, 0)

    def wait_pos(parity):
        def body(r, _):
            pltpu.make_async_copy(pos_hbm.at[0, :],
                                  pos_bufs[parity].at[pl.ds(0, D)],
                                  pos_sems[parity]).wait()
            return 0
        lax.fori_loop(0, CHUNK, body, 0)

    def start_in(c, b):
        row0 = seq0 + c * CHUNK

        def body(r, _):
            pltpu.async_copy(x_hbm.at[b, row0 + r, :],
                             x_bufs[b].at[pl.ds(r * D, D)],
                             in_sems[b])
            return 0
        lax.fori_loop(0, CHUNK, body, 0)

    def wait_in(b):
        def body(r, _):
            pltpu.make_async_copy(x_hbm.at[b, 0, :],
                                  x_bufs[b].at[pl.ds(0, D)],
                                  in_sems[b]).wait()
            return 0
        lax.fori_loop(0, CHUNK, body, 0)

    def start_out(c, b):
        row0 = seq0 + c * CHUNK

        def body(r, _):
            pltpu.async_copy(x_bufs[b].at[pl.ds(r * D, D)],
                             out_hbm.at[b, row0 + r, :],
                             out_sems[b])
            return 0
        lax.fori_loop(0, CHUNK, body, 0)

    def wait_out_start_in(c_next, b):
        def body(r, _):
            pltpu.make_async_copy(x_bufs[b].at[pl.ds(0, D)],
                                  out_hbm.at[b, 0, :],
                                  out_sems[b]).wait()

            @pl.when(c_next < N_CHUNKS)
            def _():
                pltpu.async_copy(x_hbm.at[b, seq0 + c_next * CHUNK + r, :],
                                 x_bufs[b].at[pl.ds(r * D, D)],
                                 in_sems[b])
            return 0
        lax.fori_loop(0, CHUNK, body, 0)

    def add_chunk(xb, posbuf):
        @plsc.parallel_loop(0, CHUNK_ELEMS, step=LANES, unroll=UNROLL)
        def _(off):
            plsc.addupdate(xb.at[pl.ds(off, LANES)],
                           posbuf[pl.ds(off, LANES)])

    def chunk_step(c, parity):
        @pl.when(c + 1 < N_CHUNKS)
        def _():
            start_pos(c + 1, parity ^ 1)

        wait_pos(parity)
        for b in range(BATCH):
            wait_in(b)
            add_chunk(x_bufs[b], pos_bufs[parity])
            start_out(c, b)
            if b >= 1:
                wait_out_start_in(c + 1, b - 1)
        wait_out_start_in(c + 1, BATCH - 1)

    start_pos(0, 0)
    for b in range(BATCH):
        start_in(0, b)

    def loop_body(c2, _):
        chunk_step(c2 * 2, 0)
        chunk_step(c2 * 2 + 1, 1)
        return 0

    lax.fori_loop(0, N_CHUNKS // 2, loop_body, 0)


@functools.partial(
    pl.kernel,
    mesh=plsc.VectorSubcoreMesh(core_axis_name="c", subcore_axis_name="s"),
    out_type=jax.ShapeDtypeStruct((BATCH, SEQ, D), jnp.float32),
    scratch_types=(
        [pltpu.VMEM((CHUNK_ELEMS,), jnp.float32)] * 2
        + [pltpu.VMEM((CHUNK_ELEMS,), jnp.float32)] * 4
        + [pltpu.SemaphoreType.DMA] * 10
    ),
)
def _pe_add(*refs):
    _pe_add_body(*refs)


def kernel(x, pos_table):
    return _pe_add(x, pos_table)
